# Initial kernel scaffold; baseline (speedup 1.0000x reference)
import functools

import jax
import jax.numpy as jnp
from jax import lax
from jax.experimental import pallas as pl
from jax.experimental.pallas import tpu as pltpu
from jax.experimental.pallas import tpu_sc as plsc

N = 100000
E = 3200000
HID = 32

NTILE = 16
NCORE = 2
NP = 100096
PTN = NP // NTILE
PTE = E // NTILE
CH = 2000
NCHUNK = PTE // CH
ZR = PTN // NTILE

BN = 2000
NBLK = N // BN


def _dense_tail(h, Wa_ref, ba_ref, asr_ref, adr_ref, hps_ref, al_ref,
                gmax_ref):
    hp = jnp.dot(h, Wa_ref[...], preferred_element_type=jnp.float32) \
        + ba_ref[...]
    als = jnp.sum(hp * asr_ref[...], axis=1)
    ald = jnp.sum(hp * adr_ref[...], axis=1)
    hps_ref[0] = hp[:, :16]
    hps_ref[1] = hp[:, 16:]
    al_ref[...] = jnp.stack([als, ald]).reshape(2, 1, 1, BN)
    bmax = jnp.max(als)

    @pl.when(pl.program_id(0) == 0)
    def _():
        gmax_ref[0, 0] = bmax

    @pl.when(pl.program_id(0) > 0)
    def _():
        gmax_ref[0, 0] = jnp.maximum(gmax_ref[0, 0], bmax)


def _k1_body(x_ref, W1_ref, b1_ref, W2_ref, b2_ref, Wa_ref, ba_ref,
             asr_ref, adr_ref, hps_ref, al_ref, gmax_ref):
    x = x_ref[...]
    h = jnp.maximum(x * W1_ref[...] + b1_ref[...], 0.0)
    h = jnp.maximum(
        jnp.dot(h, W2_ref[...], preferred_element_type=jnp.float32)
        + b2_ref[...], 0.0)
    _dense_tail(h, Wa_ref, ba_ref, asr_ref, adr_ref, hps_ref, al_ref,
                gmax_ref)


def _norm_elu(acc_ref, s_ref):
    o = jnp.concatenate([acc_ref[0], acc_ref[1]], axis=1)
    h = o / (s_ref[...] + 1e-16)
    return jnp.where(h > 0, h, jnp.expm1(h))


def _k2_body(acc_ref, s_ref, Wa_ref, ba_ref, asr_ref, adr_ref,
             hps_ref, al_ref, gmax_ref):
    h = _norm_elu(acc_ref, s_ref)
    _dense_tail(h, Wa_ref, ba_ref, asr_ref, adr_ref, hps_ref, al_ref,
                gmax_ref)


def _k3_body(acc_ref, s_ref, W3_ref, b3_ref, W4_ref, b4_ref, out_ref):
    h = _norm_elu(acc_ref, s_ref)
    h = jnp.maximum(
        jnp.dot(h, W3_ref[...], preferred_element_type=jnp.float32)
        + b3_ref[...], 0.0)
    z = jnp.dot(h, W4_ref[...], preferred_element_type=jnp.float32) \
        + b4_ref[...]
    zm = jnp.max(z, axis=1, keepdims=True)
    p = jnp.exp(z - zm)
    out_ref[...] = p / jnp.sum(p, axis=1, keepdims=True)


def _wspec(shape):
    n = len(shape)
    return pl.BlockSpec(shape, lambda i: (0,) * n)


_TAIL_OUT_SHAPES = [
    jax.ShapeDtypeStruct((2, N, 16), jnp.float32),
    jax.ShapeDtypeStruct((2, NBLK, 1, BN), jnp.float32),
    jax.ShapeDtypeStruct((1, 1), jnp.float32),
]
_TAIL_OUT_SPECS = [
    pl.BlockSpec((2, BN, 16), lambda i: (0, i, 0)),
    pl.BlockSpec((2, 1, 1, BN), lambda i: (0, i, 0, 0)),
    pl.BlockSpec(memory_space=pltpu.SMEM),
]
_W32 = _wspec((32, 32))
_B32 = _wspec((1, 32))


def _tc_k1(x, W1, b1, W2, b2, Wa, ba, asr, adr):
    return pl.pallas_call(
        _k1_body,
        grid=(NBLK,),
        in_specs=[pl.BlockSpec((BN, 1), lambda i: (i, 0)),
                  _wspec((1, 32)), _B32, _W32, _B32, _W32, _B32,
                  _B32, _B32],
        out_specs=_TAIL_OUT_SPECS,
        out_shape=_TAIL_OUT_SHAPES,
    )(x, W1, b1, W2, b2, Wa, ba, asr, adr)


def _tc_k2(acc, s2d, Wa, ba, asr, adr):
    return pl.pallas_call(
        _k2_body,
        grid=(NBLK,),
        in_specs=[pl.BlockSpec((2, BN, 16), lambda i: (0, i, 0)),
                  pl.BlockSpec((BN, 1), lambda i: (i, 0)),
                  _W32, _B32, _B32, _B32],
        out_specs=_TAIL_OUT_SPECS,
        out_shape=_TAIL_OUT_SHAPES,
    )(acc, s2d, Wa, ba, asr, adr)


def _tc_k3(acc, s2d, W3, b3, W4, b4):
    return pl.pallas_call(
        _k3_body,
        grid=(NBLK,),
        in_specs=[pl.BlockSpec((2, BN, 16), lambda i: (0, i, 0)),
                  pl.BlockSpec((BN, 1), lambda i: (i, 0)),
                  _W32, _B32, _wspec((32, 2)), _wspec((1, 2))],
        out_specs=pl.BlockSpec((BN, 2), lambda i: (i, 0)),
        out_shape=jax.ShapeDtypeStruct((N, 2), jnp.float32),
    )(acc, s2d, W3, b3, W4, b4)


_SC_MESH = plsc.VectorSubcoreMesh(core_axis_name="c", subcore_axis_name="s")


@functools.partial(
    pl.kernel,
    mesh=_SC_MESH,
    out_type=[jax.ShapeDtypeStruct((2 * NP, 16), jnp.float32),
              jax.ShapeDtypeStruct((NP,), jnp.float32)],
    scratch_types=[
        pltpu.VMEM((CH,), jnp.int32),
        pltpu.VMEM((CH,), jnp.int32),
        pltpu.VMEM((CH,), jnp.int32),
        pltpu.VMEM((CH,), jnp.float32),
        pltpu.VMEM((CH,), jnp.float32),
        pltpu.VMEM((CH,), jnp.float32),
        pltpu.VMEM((CH, 16), jnp.float32),
        pltpu.VMEM((PTN,), jnp.float32),
        pltpu.VMEM((ZR, 16), jnp.float32),
        pltpu.VMEM((16,), jnp.float32),
        pltpu.VMEM_SHARED((NP,), jnp.float32),
        pltpu.VMEM_SHARED((NP,), jnp.float32),
        pltpu.VMEM_SHARED((NP, 16), jnp.float32),
        pltpu.VMEM_SHARED((NP,), jnp.float32),
        pltpu.SemaphoreType.DMA,
        pltpu.SemaphoreType.DMA,
    ],
)
def _sc_att(src_hbm, dst_hbm, as_hbm, ad_hbm, g_hbm, hp_hbm,
            acc_out, s_out,
            srcv, dstv, adjv, asg, adg, exc, rows, zcol, zrow, gv,
            as_sh, ad_sh, out_sh, s_sh, sem1, sem2):
    c = lax.axis_index("c")
    t = lax.axis_index("s")
    nbase = t * PTN

    pltpu.sync_copy(as_hbm.at[pl.ds(nbase, PTN)], as_sh.at[pl.ds(nbase, PTN)])
    pltpu.sync_copy(ad_hbm.at[pl.ds(nbase, PTN)], ad_sh.at[pl.ds(nbase, PTN)])
    pltpu.sync_copy(g_hbm, gv)

    z16 = jnp.zeros((16,), jnp.float32)

    def _fill_zcol(i, _):
        zcol[pl.ds(i * 16, 16)] = z16
        return 0

    lax.fori_loop(0, PTN // 16, _fill_zcol, 0)

    def _fill_zrow(i, _):
        zrow[i, :] = z16
        return 0

    lax.fori_loop(0, ZR, _fill_zrow, 0)

    pltpu.sync_copy(zcol, s_sh.at[pl.ds(nbase, PTN)])
    for r in range(NTILE):
        pltpu.sync_copy(zrow, out_sh.at[pl.ds(nbase + r * ZR, ZR)])

    plsc.subcore_barrier()

    gvec = gv[...]
    coff = c * NP

    def chunk_body(k, _):
        ebase = pl.multiple_of(t * PTE + k * CH, 8)
        pltpu.sync_copy(src_hbm.at[pl.ds(ebase, CH)], srcv)
        pltpu.sync_copy(dst_hbm.at[pl.ds(ebase, CH)], dstv)
        cp1 = pltpu.async_copy(as_sh.at[srcv], asg, sem1)
        cp2 = pltpu.async_copy(ad_sh.at[dstv], adg, sem2)
        cp1.wait()
        cp2.wait()

        def vec_body(i, _):
            off = pl.multiple_of(i * 16, 16)
            a = asg[pl.ds(off, 16)]
            d = adg[pl.ds(off, 16)]
            z = a + d
            e = jnp.maximum(z, 0.2 * z)
            mh = gvec + d
            mh = jnp.maximum(mh, 0.2 * mh)
            exc[pl.ds(off, 16)] = jnp.exp(e - mh)
            adjv[pl.ds(off, 16)] = srcv[pl.ds(off, 16)] + coff
            return 0

        lax.fori_loop(0, CH // 16, vec_body, 0)

        @pl.when(c == 0)
        def _():
            pltpu.sync_copy(exc, s_sh.at[dstv], add=True)

        pltpu.async_copy(hp_hbm.at[adjv], rows, sem1).wait()

        def scale_body(i, _):
            b0 = i * 16
            for j in range(16):
                b = b0 + j
                exs = plsc.load_gather(exc, [jnp.full((16,), b, jnp.int32)])
                rows[b, :] = rows[b, :] * exs
            return 0

        lax.fori_loop(0, CH // 16, scale_body, 0)

        pltpu.sync_copy(rows, out_sh.at[dstv], add=True)
        return 0

    lax.fori_loop(0, NCHUNK, chunk_body, 0)

    plsc.subcore_barrier()

    pltpu.sync_copy(out_sh.at[pl.ds(nbase, PTN)],
                    acc_out.at[pl.ds(coff + nbase, PTN)])

    @pl.when(c == 0)
    def _():
        pltpu.sync_copy(s_sh.at[pl.ds(nbase, PTN)],
                        s_out.at[pl.ds(nbase, PTN)])


def _att_layer_sc(src, dst, al, gmax, hps):
    asp = jnp.pad(al[0], (0, NP - N))
    adp = jnp.pad(al[1], (0, NP - N))
    g16 = jnp.broadcast_to(gmax.reshape(1), (16,))
    hp2 = jnp.pad(hps, ((0, 0), (0, NP - N), (0, 0))).reshape(2 * NP, 16)
    acc, s = _sc_att(src, dst, asp, adp, g16, hp2)
    accN = acc.reshape(2, NP, 16)[:, :N]
    s2d = s[:N].reshape(N, 1)
    return accN, s2d


def kernel(x, edge_index, W1, b1, W2, b2, Wa, ba, a_src, a_dst,
           W3, b3, W4, b4):
    src = edge_index[0].astype(jnp.int32)
    dst = edge_index[1].astype(jnp.int32)
    b1r = b1.reshape(1, 32)
    b2r = b2.reshape(1, 32)
    bar = ba.reshape(1, 32)
    b3r = b3.reshape(1, 32)
    b4r = b4.reshape(1, 2)
    asr = a_src.reshape(1, 32)
    adr = a_dst.reshape(1, 32)

    hps, al4, gmax = _tc_k1(x, W1, b1r, W2, b2r, Wa, bar, asr, adr)
    al = al4.reshape(2, N)
    acc, s2d = _att_layer_sc(src, dst, al, gmax, hps)

    hps2, al4_2, gmax2 = _tc_k2(acc, s2d, Wa, bar, asr, adr)
    al2 = al4_2.reshape(2, N)
    acc2, s2d2 = _att_layer_sc(src, dst, al2, gmax2, hps2)

    return _tc_k3(acc2, s2d2, W3, b3r, W4, b4r)


# speedup vs baseline: 72.1418x; 72.1418x over previous
import functools

import jax
import jax.numpy as jnp
from jax import lax
from jax.experimental import pallas as pl
from jax.experimental.pallas import tpu as pltpu
from jax.experimental.pallas import tpu_sc as plsc

N = 100000
E = 3200000
HID = 32

NTILE = 16
NCORE = 2
NP = 102400
PTN = NP // NTILE
PTE = E // NTILE
CH = 400
NCHUNK = PTE // CH
NST = PTN // CH

BN = 2000
NBLK = N // BN


def _dense_tail(h, Wa_ref, ba_ref, asr_ref, adr_ref, hps_ref, al_ref,
                gmax_ref):
    hp = jnp.dot(h, Wa_ref[...], preferred_element_type=jnp.float32) \
        + ba_ref[...]
    als = jnp.sum(hp * asr_ref[...], axis=1)
    ald = jnp.sum(hp * adr_ref[...], axis=1)
    hps_ref[0] = hp[:, :16]
    hps_ref[1] = hp[:, 16:]
    al_ref[...] = jnp.stack([als, ald]).reshape(2, 1, 1, BN)
    bmax = jnp.max(als)

    @pl.when(pl.program_id(0) == 0)
    def _():
        gmax_ref[0, 0] = bmax

    @pl.when(pl.program_id(0) > 0)
    def _():
        gmax_ref[0, 0] = jnp.maximum(gmax_ref[0, 0], bmax)


def _k1_body(x_ref, W1_ref, b1_ref, W2_ref, b2_ref, Wa_ref, ba_ref,
             asr_ref, adr_ref, hps_ref, al_ref, gmax_ref):
    x = x_ref[...]
    h = jnp.maximum(x * W1_ref[...] + b1_ref[...], 0.0)
    h = jnp.maximum(
        jnp.dot(h, W2_ref[...], preferred_element_type=jnp.float32)
        + b2_ref[...], 0.0)
    _dense_tail(h, Wa_ref, ba_ref, asr_ref, adr_ref, hps_ref, al_ref,
                gmax_ref)


def _norm_elu(acc_ref, s_ref):
    o = jnp.concatenate([acc_ref[0], acc_ref[1]], axis=1)
    h = o / (s_ref[...] + 1e-16)
    return jnp.where(h > 0, h, jnp.exp(jnp.minimum(h, 0.0)) - 1.0)


def _k2_body(acc_ref, s_ref, Wa_ref, ba_ref, asr_ref, adr_ref,
             hps_ref, al_ref, gmax_ref):
    h = _norm_elu(acc_ref, s_ref)
    _dense_tail(h, Wa_ref, ba_ref, asr_ref, adr_ref, hps_ref, al_ref,
                gmax_ref)


def _k3_body(acc_ref, s_ref, W3_ref, b3_ref, W4_ref, b4_ref, out_ref):
    h = _norm_elu(acc_ref, s_ref)
    h = jnp.maximum(
        jnp.dot(h, W3_ref[...], preferred_element_type=jnp.float32)
        + b3_ref[...], 0.0)
    z = jnp.dot(h, W4_ref[...], preferred_element_type=jnp.float32) \
        + b4_ref[...]
    zm = jnp.max(z, axis=1, keepdims=True)
    p = jnp.exp(z - zm)
    out_ref[...] = p / jnp.sum(p, axis=1, keepdims=True)


def _wspec(shape):
    n = len(shape)
    return pl.BlockSpec(shape, lambda i: (0,) * n)


_TAIL_OUT_SHAPES = [
    jax.ShapeDtypeStruct((2, N, 16), jnp.float32),
    jax.ShapeDtypeStruct((2, NBLK, 1, BN), jnp.float32),
    jax.ShapeDtypeStruct((1, 1), jnp.float32),
]
_TAIL_OUT_SPECS = [
    pl.BlockSpec((2, BN, 16), lambda i: (0, i, 0)),
    pl.BlockSpec((2, 1, 1, BN), lambda i: (0, i, 0, 0)),
    pl.BlockSpec(memory_space=pltpu.SMEM),
]
_W32 = _wspec((32, 32))
_B32 = _wspec((1, 32))


def _tc_k1(x, W1, b1, W2, b2, Wa, ba, asr, adr):
    return pl.pallas_call(
        _k1_body,
        grid=(NBLK,),
        in_specs=[pl.BlockSpec((BN, 1), lambda i: (i, 0)),
                  _wspec((1, 32)), _B32, _W32, _B32, _W32, _B32,
                  _B32, _B32],
        out_specs=_TAIL_OUT_SPECS,
        out_shape=_TAIL_OUT_SHAPES,
    )(x, W1, b1, W2, b2, Wa, ba, asr, adr)


def _tc_k2(acc, s2d, Wa, ba, asr, adr):
    return pl.pallas_call(
        _k2_body,
        grid=(NBLK,),
        in_specs=[pl.BlockSpec((2, BN, 16), lambda i: (0, i, 0)),
                  pl.BlockSpec((BN, 1), lambda i: (i, 0)),
                  _W32, _B32, _B32, _B32],
        out_specs=_TAIL_OUT_SPECS,
        out_shape=_TAIL_OUT_SHAPES,
    )(acc, s2d, Wa, ba, asr, adr)


def _tc_k3(acc, s2d, W3, b3, W4, b4):
    return pl.pallas_call(
        _k3_body,
        grid=(NBLK,),
        in_specs=[pl.BlockSpec((2, BN, 16), lambda i: (0, i, 0)),
                  pl.BlockSpec((BN, 1), lambda i: (i, 0)),
                  _W32, _B32, _wspec((32, 2)), _wspec((1, 2))],
        out_specs=pl.BlockSpec((BN, 2), lambda i: (i, 0)),
        out_shape=jax.ShapeDtypeStruct((N, 2), jnp.float32),
    )(acc, s2d, W3, b3, W4, b4)


_SC_MESH = plsc.VectorSubcoreMesh(core_axis_name="c", subcore_axis_name="s")


@functools.partial(
    pl.kernel,
    mesh=_SC_MESH,
    compiler_params=pltpu.CompilerParams(use_tc_tiling_on_sc=False),
    out_type=[jax.ShapeDtypeStruct((2 * NP, 16), jnp.float32),
              jax.ShapeDtypeStruct((NP,), jnp.float32)],
    scratch_types=[
        pltpu.VMEM((CH,), jnp.int32),
        pltpu.VMEM((CH,), jnp.int32),
        pltpu.VMEM((CH,), jnp.int32),
        pltpu.VMEM((CH,), jnp.float32),
        pltpu.VMEM((CH,), jnp.float32),
        pltpu.VMEM((CH,), jnp.float32),
        pltpu.VMEM((CH, 16), jnp.float32),
        pltpu.VMEM((16,), jnp.float32),
        pltpu.VMEM_SHARED((NP,), jnp.float32),
        pltpu.VMEM_SHARED((NP,), jnp.float32),
        pltpu.VMEM_SHARED((NP, 16), jnp.float32),
        pltpu.VMEM_SHARED((NP,), jnp.float32),
        pltpu.SemaphoreType.DMA,
        pltpu.SemaphoreType.DMA,
        pltpu.SemaphoreType.DMA,
    ],
)
def _sc_att(src_hbm, dst_hbm, as_hbm, ad_hbm, g_hbm, hp_hbm,
            acc_out, s_out,
            srcv, dstv, adjv, asg, adg, stage, rows, gv,
            as_sh, ad_sh, out_sh, s_sh, sem1, sem2, sem3):
    c = lax.axis_index("c")
    t = lax.axis_index("s")
    nbase = t * PTN
    z16 = jnp.zeros((16,), jnp.float32)

    def stage_body(j, _):
        off = nbase + j * CH
        pltpu.sync_copy(as_hbm.at[pl.ds(off, CH)], stage)
        pltpu.sync_copy(stage, as_sh.at[pl.ds(off, CH)])
        pltpu.sync_copy(ad_hbm.at[pl.ds(off, CH)], stage)
        pltpu.sync_copy(stage, ad_sh.at[pl.ds(off, CH)])
        return 0

    lax.fori_loop(0, NST, stage_body, 0)
    pltpu.sync_copy(g_hbm, gv)

    def _fill_stage(i, _):
        stage[pl.ds(i * 16, 16)] = z16
        return 0

    lax.fori_loop(0, CH // 16, _fill_stage, 0)

    def _fill_rows(i, _):
        rows[i, :] = z16
        return 0

    lax.fori_loop(0, CH, _fill_rows, 0)

    def zero_body(j, _):
        off = nbase + j * CH
        pltpu.sync_copy(stage, s_sh.at[pl.ds(off, CH)])
        pltpu.sync_copy(rows, out_sh.at[pl.ds(off, CH)])
        return 0

    lax.fori_loop(0, NST, zero_body, 0)

    plsc.subcore_barrier()

    gvec = gv[...]
    coff = c * NP

    def chunk_body(k, _):
        ebase = pl.multiple_of(t * PTE + k * CH, 8)
        pltpu.sync_copy(src_hbm.at[pl.ds(ebase, CH)], srcv)
        pltpu.sync_copy(dst_hbm.at[pl.ds(ebase, CH)], dstv)
        cp1 = pltpu.async_copy(as_sh.at[srcv], asg, sem1)
        cp2 = pltpu.async_copy(ad_sh.at[dstv], adg, sem2)

        def adj_body(i, _):
            off = pl.multiple_of(i * 16, 16)
            adjv[pl.ds(off, 16)] = srcv[pl.ds(off, 16)] + coff
            return 0

        lax.fori_loop(0, CH // 16, adj_body, 0)
        cp3 = pltpu.async_copy(hp_hbm.at[adjv], rows, sem3)
        cp1.wait()
        cp2.wait()

        def vec_body(i, _):
            off = pl.multiple_of(i * 16, 16)
            a = asg[pl.ds(off, 16)]
            d = adg[pl.ds(off, 16)]
            z = a + d
            e = jnp.maximum(z, 0.2 * z)
            mh = gvec + d
            mh = jnp.maximum(mh, 0.2 * mh)
            asg[pl.ds(off, 16)] = jnp.exp(e - mh)
            return 0

        lax.fori_loop(0, CH // 16, vec_body, 0)

        @pl.when(c == 0)
        def _():
            pltpu.sync_copy(asg, s_sh.at[dstv], add=True)

        cp3.wait()

        def scale_body(i, _):
            b0 = i * 16
            exv = asg[pl.ds(pl.multiple_of(b0, 16), 16)]
            for j in range(16):
                b = b0 + j
                rows[b, :] = rows[b, :] * exv[j]
            return 0

        lax.fori_loop(0, CH // 16, scale_body, 0)

        pltpu.sync_copy(rows, out_sh.at[dstv], add=True)
        return 0

    lax.fori_loop(0, NCHUNK, chunk_body, 0)

    plsc.subcore_barrier()

    def flush_body(q, _):
        off = nbase + q * CH
        pltpu.sync_copy(out_sh.at[pl.ds(off, CH)], rows)
        pltpu.sync_copy(rows, acc_out.at[pl.ds(coff + off, CH)])
        return 0

    lax.fori_loop(0, NST, flush_body, 0)

    @pl.when(c == 0)
    def _():
        def sflush_body(q, _):
            off = nbase + q * CH
            pltpu.sync_copy(s_sh.at[pl.ds(off, CH)], stage)
            pltpu.sync_copy(stage, s_out.at[pl.ds(off, CH)])
            return 0

        lax.fori_loop(0, NST, sflush_body, 0)


def _att_layer_sc(src, dst, al, gmax, hps):
    asp = jnp.pad(al[0], (0, NP - N))
    adp = jnp.pad(al[1], (0, NP - N))
    g16 = jnp.broadcast_to(gmax.reshape(1), (16,))
    hp2 = jnp.pad(hps, ((0, 0), (0, NP - N), (0, 0))).reshape(2 * NP, 16)
    acc, s = _sc_att(src, dst, asp, adp, g16, hp2)
    accN = acc.reshape(2, NP, 16)[:, :N]
    s2d = s[:N].reshape(N, 1)
    return accN, s2d


def kernel(x, edge_index, W1, b1, W2, b2, Wa, ba, a_src, a_dst,
           W3, b3, W4, b4):
    src = edge_index[0].astype(jnp.int32)
    dst = edge_index[1].astype(jnp.int32)
    b1r = b1.reshape(1, 32)
    b2r = b2.reshape(1, 32)
    bar = ba.reshape(1, 32)
    b3r = b3.reshape(1, 32)
    b4r = b4.reshape(1, 2)
    asr = a_src.reshape(1, 32)
    adr = a_dst.reshape(1, 32)

    hps, al4, gmax = _tc_k1(x, W1, b1r, W2, b2r, Wa, bar, asr, adr)
    al = al4.reshape(2, N)
    acc, s2d = _att_layer_sc(src, dst, al, gmax, hps)

    hps2, al4_2, gmax2 = _tc_k2(acc, s2d, Wa, bar, asr, adr)
    al2 = al4_2.reshape(2, N)
    acc2, s2d2 = _att_layer_sc(src, dst, al2, gmax2, hps2)

    return _tc_k3(acc2, s2d2, W3, b3r, W4, b4r)


# speedup vs baseline: 77.4153x; 1.0731x over previous
import functools

import jax
import jax.numpy as jnp
from jax import lax
from jax.experimental import pallas as pl
from jax.experimental.pallas import tpu as pltpu
from jax.experimental.pallas import tpu_sc as plsc

N = 100000
E = 3200000
HID = 32

NTILE = 16
NCORE = 2
NP = 102400
PTN = NP // NTILE
PTE = E // NTILE
CH = 400
NCHUNK = PTE // CH
NST = PTN // CH

BN = 2000
NBLK = N // BN
BNP = 2048
NBLKP = NP // BNP


def _dense_tail(h, Wa_ref, ba_ref, asr_ref, adr_ref, hps_ref, al_ref,
                gmax_ref):
    hp = jnp.dot(h, Wa_ref[...], preferred_element_type=jnp.float32) \
        + ba_ref[...]
    als = jnp.sum(hp * asr_ref[...], axis=1)
    ald = jnp.sum(hp * adr_ref[...], axis=1)
    hps_ref[0] = hp[:, :16]
    hps_ref[1] = hp[:, 16:]
    al_ref[...] = jnp.stack([als, ald]).reshape(2, 1, 1, BNP)
    bmax = jnp.max(als)

    @pl.when(pl.program_id(0) == 0)
    def _():
        gmax_ref[0, 0] = bmax

    @pl.when(pl.program_id(0) > 0)
    def _():
        gmax_ref[0, 0] = jnp.maximum(gmax_ref[0, 0], bmax)


def _k1_body(x_ref, W1_ref, b1_ref, W2_ref, b2_ref, Wa_ref, ba_ref,
             asr_ref, adr_ref, hps_ref, al_ref, gmax_ref):
    x = x_ref[...]
    h = jnp.maximum(x * W1_ref[...] + b1_ref[...], 0.0)
    h = jnp.maximum(
        jnp.dot(h, W2_ref[...], preferred_element_type=jnp.float32)
        + b2_ref[...], 0.0)
    _dense_tail(h, Wa_ref, ba_ref, asr_ref, adr_ref, hps_ref, al_ref,
                gmax_ref)


def _norm_elu(acc_ref, s_ref):
    o = jnp.concatenate([acc_ref[0], acc_ref[1]], axis=1)
    h = o / (s_ref[...] + 1e-16)
    return jnp.where(h > 0, h, jnp.exp(jnp.minimum(h, 0.0)) - 1.0)


def _k2_body(acc_ref, s_ref, Wa_ref, ba_ref, asr_ref, adr_ref,
             hps_ref, al_ref, gmax_ref):
    h = _norm_elu(acc_ref, s_ref)
    _dense_tail(h, Wa_ref, ba_ref, asr_ref, adr_ref, hps_ref, al_ref,
                gmax_ref)


def _k3_body(acc_ref, s_ref, W3_ref, b3_ref, W4_ref, b4_ref, out_ref):
    h = _norm_elu(acc_ref, s_ref)
    h = jnp.maximum(
        jnp.dot(h, W3_ref[...], preferred_element_type=jnp.float32)
        + b3_ref[...], 0.0)
    z = jnp.dot(h, W4_ref[...], preferred_element_type=jnp.float32) \
        + b4_ref[...]
    zm = jnp.max(z, axis=1, keepdims=True)
    p = jnp.exp(z - zm)
    out_ref[...] = p / jnp.sum(p, axis=1, keepdims=True)


def _wspec(shape):
    n = len(shape)
    return pl.BlockSpec(shape, lambda i: (0,) * n)


_TAIL_OUT_SHAPES = [
    jax.ShapeDtypeStruct((2, NP, 16), jnp.float32),
    jax.ShapeDtypeStruct((2, NBLKP, 1, BNP), jnp.float32),
    jax.ShapeDtypeStruct((1, 1), jnp.float32),
]
_TAIL_OUT_SPECS = [
    pl.BlockSpec((2, BNP, 16), lambda i: (0, i, 0)),
    pl.BlockSpec((2, 1, 1, BNP), lambda i: (0, i, 0, 0)),
    pl.BlockSpec(memory_space=pltpu.SMEM),
]
_W32 = _wspec((32, 32))
_B32 = _wspec((1, 32))


def _tc_k1(x, W1, b1, W2, b2, Wa, ba, asr, adr):
    return pl.pallas_call(
        _k1_body,
        grid=(NBLKP,),
        in_specs=[pl.BlockSpec((BNP, 1), lambda i: (i, 0)),
                  _wspec((1, 32)), _B32, _W32, _B32, _W32, _B32,
                  _B32, _B32],
        out_specs=_TAIL_OUT_SPECS,
        out_shape=_TAIL_OUT_SHAPES,
    )(x, W1, b1, W2, b2, Wa, ba, asr, adr)


def _tc_k2(acc, s2d, Wa, ba, asr, adr):
    return pl.pallas_call(
        _k2_body,
        grid=(NBLKP,),
        in_specs=[pl.BlockSpec((2, BNP, 16), lambda i: (0, i, 0)),
                  pl.BlockSpec((BNP, 1), lambda i: (i, 0)),
                  _W32, _B32, _B32, _B32],
        out_specs=_TAIL_OUT_SPECS,
        out_shape=_TAIL_OUT_SHAPES,
    )(acc, s2d, Wa, ba, asr, adr)


def _tc_k3(acc, s2d, W3, b3, W4, b4):
    return pl.pallas_call(
        _k3_body,
        grid=(NBLK,),
        in_specs=[pl.BlockSpec((2, BN, 16), lambda i: (0, i, 0)),
                  pl.BlockSpec((BN, 1), lambda i: (i, 0)),
                  _W32, _B32, _wspec((32, 2)), _wspec((1, 2))],
        out_specs=pl.BlockSpec((BN, 2), lambda i: (i, 0)),
        out_shape=jax.ShapeDtypeStruct((N, 2), jnp.float32),
    )(acc, s2d, W3, b3, W4, b4)


_SC_MESH = plsc.VectorSubcoreMesh(core_axis_name="c", subcore_axis_name="s")


@functools.partial(
    pl.kernel,
    mesh=_SC_MESH,
    compiler_params=pltpu.CompilerParams(use_tc_tiling_on_sc=False),
    out_type=[jax.ShapeDtypeStruct((2 * NP, 16), jnp.float32),
              jax.ShapeDtypeStruct((NP,), jnp.float32)],
    scratch_types=[
        pltpu.VMEM((CH,), jnp.int32),
        pltpu.VMEM((CH,), jnp.int32),
        pltpu.VMEM((CH,), jnp.int32),
        pltpu.VMEM((CH,), jnp.float32),
        pltpu.VMEM((CH,), jnp.float32),
        pltpu.VMEM((CH,), jnp.float32),
        pltpu.VMEM((CH, 16), jnp.float32),
        pltpu.VMEM((16,), jnp.float32),
        pltpu.VMEM_SHARED((NP,), jnp.float32),
        pltpu.VMEM_SHARED((NP,), jnp.float32),
        pltpu.VMEM_SHARED((NP, 16), jnp.float32),
        pltpu.VMEM_SHARED((NP,), jnp.float32),
        pltpu.SemaphoreType.DMA,
        pltpu.SemaphoreType.DMA,
        pltpu.SemaphoreType.DMA,
    ],
)
def _sc_att(src_hbm, dst_hbm, as_hbm, ad_hbm, g_hbm, hp_hbm,
            acc_out, s_out,
            srcv, dstv, adjv, asg, adg, stage, rows, gv,
            as_sh, ad_sh, out_sh, s_sh, sem1, sem2, sem3):
    c = lax.axis_index("c")
    t = lax.axis_index("s")
    nbase = t * PTN
    z16 = jnp.zeros((16,), jnp.float32)

    def stage_body(j, _):
        off = nbase + j * CH
        pltpu.sync_copy(as_hbm.at[pl.ds(off, CH)], stage)
        pltpu.sync_copy(stage, as_sh.at[pl.ds(off, CH)])
        pltpu.sync_copy(ad_hbm.at[pl.ds(off, CH)], stage)
        pltpu.sync_copy(stage, ad_sh.at[pl.ds(off, CH)])
        return 0

    lax.fori_loop(0, NST, stage_body, 0)
    pltpu.sync_copy(g_hbm, gv)

    def _fill_stage(i, _):
        stage[pl.ds(i * 16, 16)] = z16
        return 0

    lax.fori_loop(0, CH // 16, _fill_stage, 0)

    def _fill_rows(i, _):
        rows[i, :] = z16
        return 0

    lax.fori_loop(0, CH, _fill_rows, 0)

    def zero_body(j, _):
        off = nbase + j * CH
        pltpu.sync_copy(stage, s_sh.at[pl.ds(off, CH)])
        pltpu.sync_copy(rows, out_sh.at[pl.ds(off, CH)])
        return 0

    lax.fori_loop(0, NST, zero_body, 0)

    plsc.subcore_barrier()

    gvec = gv[...]
    coff = c * NP

    def chunk_body(k, _):
        ebase = pl.multiple_of(t * PTE + k * CH, 8)
        pltpu.sync_copy(src_hbm.at[pl.ds(ebase, CH)], srcv)
        pltpu.sync_copy(dst_hbm.at[pl.ds(ebase, CH)], dstv)
        cp1 = pltpu.async_copy(as_sh.at[srcv], asg, sem1)
        cp2 = pltpu.async_copy(ad_sh.at[dstv], adg, sem2)

        def adj_body(i, _):
            off = pl.multiple_of(i * 16, 16)
            adjv[pl.ds(off, 16)] = srcv[pl.ds(off, 16)] + coff
            return 0

        lax.fori_loop(0, CH // 16, adj_body, 0)
        cp3 = pltpu.async_copy(hp_hbm.at[adjv], rows, sem3)
        cp1.wait()
        cp2.wait()

        def vec_body(i, _):
            off = pl.multiple_of(i * 16, 16)
            a = asg[pl.ds(off, 16)]
            d = adg[pl.ds(off, 16)]
            z = a + d
            e = jnp.maximum(z, 0.2 * z)
            mh = gvec + d
            mh = jnp.maximum(mh, 0.2 * mh)
            asg[pl.ds(off, 16)] = jnp.exp(e - mh)
            return 0

        lax.fori_loop(0, CH // 16, vec_body, 0)

        @pl.when(c == 0)
        def _():
            pltpu.sync_copy(asg, s_sh.at[dstv], add=True)

        cp3.wait()

        def scale_body(i, _):
            b0 = i * 16
            exv = asg[pl.ds(pl.multiple_of(b0, 16), 16)]
            for j in range(16):
                b = b0 + j
                rows[b, :] = rows[b, :] * exv[j]
            return 0

        lax.fori_loop(0, CH // 16, scale_body, 0)

        pltpu.sync_copy(rows, out_sh.at[dstv], add=True)
        return 0

    lax.fori_loop(0, NCHUNK, chunk_body, 0)

    plsc.subcore_barrier()

    def flush_body(q, _):
        off = nbase + q * CH
        pltpu.sync_copy(out_sh.at[pl.ds(off, CH)], rows)
        pltpu.sync_copy(rows, acc_out.at[pl.ds(coff + off, CH)])
        return 0

    lax.fori_loop(0, NST, flush_body, 0)

    @pl.when(c == 0)
    def _():
        def sflush_body(q, _):
            off = nbase + q * CH
            pltpu.sync_copy(s_sh.at[pl.ds(off, CH)], stage)
            pltpu.sync_copy(stage, s_out.at[pl.ds(off, CH)])
            return 0

        lax.fori_loop(0, NST, sflush_body, 0)


def _att_layer_sc(src, dst, al, gmax, hps):
    g16 = jnp.broadcast_to(gmax.reshape(1), (16,))
    hp2 = hps.reshape(2 * NP, 16)
    acc, s = _sc_att(src, dst, al[0], al[1], g16, hp2)
    return acc.reshape(2, NP, 16), s.reshape(NP, 1)


def kernel(x, edge_index, W1, b1, W2, b2, Wa, ba, a_src, a_dst,
           W3, b3, W4, b4):
    src = edge_index[0].astype(jnp.int32)
    dst = edge_index[1].astype(jnp.int32)
    xp = jnp.pad(x, ((0, NP - N), (0, 0)))
    b1r = b1.reshape(1, 32)
    b2r = b2.reshape(1, 32)
    bar = ba.reshape(1, 32)
    b3r = b3.reshape(1, 32)
    b4r = b4.reshape(1, 2)
    asr = a_src.reshape(1, 32)
    adr = a_dst.reshape(1, 32)

    hps, al4, gmax = _tc_k1(xp, W1, b1r, W2, b2r, Wa, bar, asr, adr)
    al = al4.reshape(2, NP)
    acc, s2d = _att_layer_sc(src, dst, al, gmax, hps)

    hps2, al4_2, gmax2 = _tc_k2(acc, s2d, Wa, bar, asr, adr)
    al2 = al4_2.reshape(2, NP)
    acc2, s2d2 = _att_layer_sc(src, dst, al2, gmax2, hps2)

    return _tc_k3(acc2, s2d2, W3, b3r, W4, b4r)


# speedup vs baseline: 106.4751x; 1.3754x over previous
import functools

import jax
import jax.numpy as jnp
from jax import lax
from jax.experimental import pallas as pl
from jax.experimental.pallas import tpu as pltpu
from jax.experimental.pallas import tpu_sc as plsc

N = 100000
E = 3200000
HID = 32

NTILE = 16
NCORE = 2
NP = 102400
PTN = NP // NTILE
PTE = E // NTILE
CH = 400
NCHUNK = PTE // CH
NST = PTN // CH

BN = 2000
NBLK = N // BN
BNP = 2048
NBLKP = NP // BNP


def _dense_tail(h, Wa_ref, ba_ref, asr_ref, adr_ref, hps_ref, al_ref,
                gmax_ref):
    hp = jnp.dot(h, Wa_ref[...], preferred_element_type=jnp.float32) \
        + ba_ref[...]
    als = jnp.sum(hp * asr_ref[...], axis=1)
    ald = jnp.sum(hp * adr_ref[...], axis=1)
    hps_ref[0] = hp[:, :16]
    hps_ref[1] = hp[:, 16:]
    al_ref[...] = jnp.stack([als, ald]).reshape(2, 1, 1, BNP)
    bmax = jnp.max(als)

    @pl.when(pl.program_id(0) == 0)
    def _():
        gmax_ref[0, 0] = bmax

    @pl.when(pl.program_id(0) > 0)
    def _():
        gmax_ref[0, 0] = jnp.maximum(gmax_ref[0, 0], bmax)


def _k1_body(x_ref, W1_ref, b1_ref, W2_ref, b2_ref, Wa_ref, ba_ref,
             asr_ref, adr_ref, hps_ref, al_ref, gmax_ref):
    x = x_ref[...]
    h = jnp.maximum(x * W1_ref[...] + b1_ref[...], 0.0)
    h = jnp.maximum(
        jnp.dot(h, W2_ref[...], preferred_element_type=jnp.float32)
        + b2_ref[...], 0.0)
    _dense_tail(h, Wa_ref, ba_ref, asr_ref, adr_ref, hps_ref, al_ref,
                gmax_ref)


def _norm_elu(acc_ref, s_ref):
    o = jnp.concatenate([acc_ref[0], acc_ref[1]], axis=1)
    h = o / (s_ref[...] + 1e-16)
    return jnp.where(h > 0, h, jnp.exp(jnp.minimum(h, 0.0)) - 1.0)


def _k2_body(acc_ref, s_ref, Wa_ref, ba_ref, asr_ref, adr_ref,
             hps_ref, al_ref, gmax_ref):
    h = _norm_elu(acc_ref, s_ref)
    _dense_tail(h, Wa_ref, ba_ref, asr_ref, adr_ref, hps_ref, al_ref,
                gmax_ref)


def _k3_body(acc_ref, s_ref, W3_ref, b3_ref, W4_ref, b4_ref, out_ref):
    h = _norm_elu(acc_ref, s_ref)
    h = jnp.maximum(
        jnp.dot(h, W3_ref[...], preferred_element_type=jnp.float32)
        + b3_ref[...], 0.0)
    z = jnp.dot(h, W4_ref[...], preferred_element_type=jnp.float32) \
        + b4_ref[...]
    zm = jnp.max(z, axis=1, keepdims=True)
    p = jnp.exp(z - zm)
    out_ref[...] = p / jnp.sum(p, axis=1, keepdims=True)


def _wspec(shape):
    n = len(shape)
    return pl.BlockSpec(shape, lambda i: (0,) * n)


_TAIL_OUT_SHAPES = [
    jax.ShapeDtypeStruct((2, NP, 16), jnp.float32),
    jax.ShapeDtypeStruct((2, NBLKP, 1, BNP), jnp.float32),
    jax.ShapeDtypeStruct((1, 1), jnp.float32),
]
_TAIL_OUT_SPECS = [
    pl.BlockSpec((2, BNP, 16), lambda i: (0, i, 0)),
    pl.BlockSpec((2, 1, 1, BNP), lambda i: (0, i, 0, 0)),
    pl.BlockSpec(memory_space=pltpu.SMEM),
]
_W32 = _wspec((32, 32))
_B32 = _wspec((1, 32))


def _tc_k1(x, W1, b1, W2, b2, Wa, ba, asr, adr):
    return pl.pallas_call(
        _k1_body,
        grid=(NBLKP,),
        in_specs=[pl.BlockSpec((BNP, 1), lambda i: (i, 0)),
                  _wspec((1, 32)), _B32, _W32, _B32, _W32, _B32,
                  _B32, _B32],
        out_specs=_TAIL_OUT_SPECS,
        out_shape=_TAIL_OUT_SHAPES,
    )(x, W1, b1, W2, b2, Wa, ba, asr, adr)


def _tc_k2(acc, s2d, Wa, ba, asr, adr):
    return pl.pallas_call(
        _k2_body,
        grid=(NBLKP,),
        in_specs=[pl.BlockSpec((2, BNP, 16), lambda i: (0, i, 0)),
                  pl.BlockSpec((BNP, 1), lambda i: (i, 0)),
                  _W32, _B32, _B32, _B32],
        out_specs=_TAIL_OUT_SPECS,
        out_shape=_TAIL_OUT_SHAPES,
    )(acc, s2d, Wa, ba, asr, adr)


def _tc_k3(acc, s2d, W3, b3, W4, b4):
    return pl.pallas_call(
        _k3_body,
        grid=(NBLK,),
        in_specs=[pl.BlockSpec((2, BN, 16), lambda i: (0, i, 0)),
                  pl.BlockSpec((BN, 1), lambda i: (i, 0)),
                  _W32, _B32, _wspec((32, 2)), _wspec((1, 2))],
        out_specs=pl.BlockSpec((BN, 2), lambda i: (i, 0)),
        out_shape=jax.ShapeDtypeStruct((N, 2), jnp.float32),
    )(acc, s2d, W3, b3, W4, b4)


_SC_MESH = plsc.VectorSubcoreMesh(core_axis_name="c", subcore_axis_name="s")


@functools.partial(
    pl.kernel,
    mesh=_SC_MESH,
    compiler_params=pltpu.CompilerParams(use_tc_tiling_on_sc=False),
    out_type=[jax.ShapeDtypeStruct((2 * NP, 16), jnp.float32),
              jax.ShapeDtypeStruct((NP,), jnp.float32)],
    scratch_types=[
        pltpu.VMEM((CH,), jnp.int32),
        pltpu.VMEM((CH,), jnp.int32),
        pltpu.VMEM((CH,), jnp.int32),
        pltpu.VMEM((CH,), jnp.int32),
        pltpu.VMEM((CH,), jnp.int32),
        pltpu.VMEM((CH,), jnp.float32),
        pltpu.VMEM((CH,), jnp.float32),
        pltpu.VMEM((CH, 16), jnp.float32),
        pltpu.VMEM((16,), jnp.float32),
        pltpu.VMEM_SHARED((NP,), jnp.float32),
        pltpu.VMEM_SHARED((NP,), jnp.float32),
        pltpu.VMEM_SHARED((NP, 16), jnp.float32),
        pltpu.VMEM_SHARED((NP,), jnp.float32),
        pltpu.SemaphoreType.DMA,
        pltpu.SemaphoreType.DMA,
        pltpu.SemaphoreType.DMA,
        pltpu.SemaphoreType.DMA,
        pltpu.SemaphoreType.DMA,
        pltpu.SemaphoreType.DMA,
        pltpu.SemaphoreType.DMA,
    ],
)
def _sc_att(src_hbm, dst_hbm, as_hbm, ad_hbm, g_hbm, hp_hbm,
            acc_out, s_out,
            srcv, dstv, src2, dst2, adjv, asg, adg, rows, gv,
            as_sh, ad_sh, out_sh, s_sh,
            sem1, sem2, sem3, semA, semB, semC, semD):
    c = lax.axis_index("c")
    t = lax.axis_index("s")
    nbase = t * PTN
    z16 = jnp.zeros((16,), jnp.float32)
    stage = asg

    def stage_body(j, _):
        off = nbase + j * CH
        pltpu.sync_copy(as_hbm.at[pl.ds(off, CH)], stage)
        pltpu.sync_copy(stage, as_sh.at[pl.ds(off, CH)])
        pltpu.sync_copy(ad_hbm.at[pl.ds(off, CH)], stage)
        pltpu.sync_copy(stage, ad_sh.at[pl.ds(off, CH)])
        return 0

    lax.fori_loop(0, NST, stage_body, 0)
    pltpu.sync_copy(g_hbm, gv)

    def _fill_stage(i, _):
        stage[pl.ds(i * 16, 16)] = z16
        return 0

    lax.fori_loop(0, CH // 16, _fill_stage, 0)

    def _fill_rows(i, _):
        rows[i, :] = z16
        return 0

    lax.fori_loop(0, CH, _fill_rows, 0)

    def zero_body(j, _):
        off = nbase + j * CH
        pltpu.sync_copy(stage, s_sh.at[pl.ds(off, CH)])
        pltpu.sync_copy(rows, out_sh.at[pl.ds(off, CH)])
        return 0

    lax.fori_loop(0, NST, zero_body, 0)

    plsc.subcore_barrier()

    gvec = gv[...]
    coff = c * NP
    ebase0 = pl.multiple_of(t * PTE, 8)

    def _eb(k):
        return pl.multiple_of(t * PTE + k * CH, 8)

    def process(sv, dv):
        cp1 = pltpu.async_copy(as_sh.at[sv], asg, sem1)
        cp2 = pltpu.async_copy(ad_sh.at[dv], adg, sem2)

        def adj_body(i, _):
            off = pl.multiple_of(i * 16, 16)
            adjv[pl.ds(off, 16)] = sv[pl.ds(off, 16)] + coff
            return 0

        lax.fori_loop(0, CH // 16, adj_body, 0)
        cp3 = pltpu.async_copy(hp_hbm.at[adjv], rows, sem3)
        cp1.wait()
        cp2.wait()

        def vec_body(i, _):
            off = pl.multiple_of(i * 16, 16)
            a = asg[pl.ds(off, 16)]
            d = adg[pl.ds(off, 16)]
            z = a + d
            e = jnp.maximum(z, 0.2 * z)
            mh = gvec + d
            mh = jnp.maximum(mh, 0.2 * mh)
            asg[pl.ds(off, 16)] = jnp.exp(e - mh)
            return 0

        lax.fori_loop(0, CH // 16, vec_body, 0)

        @pl.when(c == 0)
        def _():
            pltpu.sync_copy(asg, s_sh.at[dv], add=True)

        cp3.wait()

        def scale_body(i, _):
            b0 = i * 16
            exv = asg[pl.ds(pl.multiple_of(b0, 16), 16)]
            for j in range(16):
                b = b0 + j
                rows[b, :] = rows[b, :] * exv[j]
            return 0

        lax.fori_loop(0, CH // 16, scale_body, 0)

        pltpu.sync_copy(rows, out_sh.at[dv], add=True)

    pltpu.async_copy(src_hbm.at[pl.ds(ebase0, CH)], srcv, semA)
    pltpu.async_copy(dst_hbm.at[pl.ds(ebase0, CH)], dstv, semB)

    def pair_body(p, _):
        ebA = _eb(2 * p)
        ebB = _eb(2 * p + 1)
        pltpu.make_async_copy(src_hbm.at[pl.ds(ebA, CH)], srcv, semA).wait()
        pltpu.make_async_copy(dst_hbm.at[pl.ds(ebA, CH)], dstv, semB).wait()
        pltpu.async_copy(src_hbm.at[pl.ds(ebB, CH)], src2, semC)
        pltpu.async_copy(dst_hbm.at[pl.ds(ebB, CH)], dst2, semD)
        process(srcv, dstv)
        pltpu.make_async_copy(src_hbm.at[pl.ds(ebB, CH)], src2, semC).wait()
        pltpu.make_async_copy(dst_hbm.at[pl.ds(ebB, CH)], dst2, semD).wait()

        @pl.when(p < NCHUNK // 2 - 1)
        def _():
            ebN = _eb(2 * p + 2)
            pltpu.async_copy(src_hbm.at[pl.ds(ebN, CH)], srcv, semA)
            pltpu.async_copy(dst_hbm.at[pl.ds(ebN, CH)], dstv, semB)

        process(src2, dst2)
        return 0

    lax.fori_loop(0, NCHUNK // 2, pair_body, 0)

    plsc.subcore_barrier()

    def flush_body(q, _):
        off = nbase + q * CH
        pltpu.sync_copy(out_sh.at[pl.ds(off, CH)], rows)
        pltpu.sync_copy(rows, acc_out.at[pl.ds(coff + off, CH)])
        return 0

    lax.fori_loop(0, NST, flush_body, 0)

    @pl.when(c == 0)
    def _():
        def sflush_body(q, _):
            off = nbase + q * CH
            pltpu.sync_copy(s_sh.at[pl.ds(off, CH)], stage)
            pltpu.sync_copy(stage, s_out.at[pl.ds(off, CH)])
            return 0

        lax.fori_loop(0, NST, sflush_body, 0)


def _att_layer_sc(src, dst, al, gmax, hps):
    g16 = jnp.broadcast_to(gmax.reshape(1), (16,))
    hp2 = hps.reshape(2 * NP, 16)
    acc, s = _sc_att(src, dst, al[0], al[1], g16, hp2)
    return acc.reshape(2, NP, 16), s.reshape(NP, 1)


def kernel(x, edge_index, W1, b1, W2, b2, Wa, ba, a_src, a_dst,
           W3, b3, W4, b4):
    src = edge_index[0].astype(jnp.int32)
    dst = edge_index[1].astype(jnp.int32)
    xp = jnp.pad(x, ((0, NP - N), (0, 0)))
    b1r = b1.reshape(1, 32)
    b2r = b2.reshape(1, 32)
    bar = ba.reshape(1, 32)
    b3r = b3.reshape(1, 32)
    b4r = b4.reshape(1, 2)
    asr = a_src.reshape(1, 32)
    adr = a_dst.reshape(1, 32)

    hps, al4, gmax = _tc_k1(xp, W1, b1r, W2, b2r, Wa, bar, asr, adr)
    al = al4.reshape(2, NP)
    acc, s2d = _att_layer_sc(src, dst, al, gmax, hps)

    hps2, al4_2, gmax2 = _tc_k2(acc, s2d, Wa, bar, asr, adr)
    al2 = al4_2.reshape(2, NP)
    acc2, s2d2 = _att_layer_sc(src, dst, al2, gmax2, hps2)

    return _tc_k3(acc2, s2d2, W3, b3r, W4, b4r)
